# baseline (speedup 1.0000x reference)
import jax
import jax.numpy as jnp
import numpy as np
from jax.experimental import pallas as pl
from jax.experimental.pallas import tpu as pltpu

H = 1024
DFF = 4096
E = 8
NB = 2
B = 32
S = 32

SEQ_PER_BLK = 8
NUM_BLOCKS = E + (NB * B - E) // SEQ_PER_BLK
NUM_SLOTS = NUM_BLOCKS * SEQ_PER_BLK
TRASH_ROW = NB * B


DFFT = 2048
NT = DFF // DFFT


def _ffn_body(nreal, e_of_blk, seq_ids, pos_ids, x_ref, w1_ref, b1_ref,
              w2_ref, b2_ref, wgt_ref, out_ref):
    t = pl.program_id(0)
    b = pl.program_id(1)

    @pl.when(b < nreal[0])
    def _work():
        xs = [x_ref[seq_ids[SEQ_PER_BLK * b + i]] for i in range(SEQ_PER_BLK)]
        xg = jnp.concatenate(xs, axis=0)
        h = jnp.dot(xg, w1_ref[0], preferred_element_type=jnp.float32)
        h = h + b1_ref[0]
        h = 0.5 * h * (1.0 + jax.lax.erf(
            h * (1.0 / np.sqrt(2.0).astype(np.float32))))
        c = jnp.dot(h, w2_ref[0], preferred_element_type=jnp.float32)
        b2v = b2_ref[0]
        for i in range(SEQ_PER_BLK):
            pos = pos_ids[SEQ_PER_BLK * b + i]
            w = wgt_ref[0, 0, i]
            val = w * (c[S * i:S * (i + 1), :] + b2v)

            @pl.when(t == 0)
            def _init():
                out_ref[pl.ds(pos, 1)] = val[None]

            @pl.when(t != 0)
            def _acc():
                out_ref[pl.ds(pos, 1)] += (w * c[S * i:S * (i + 1), :])[None]


def _run_ffn(x, W1, b1, W2, b2, nreal, e_of_blk, seq_ids, pos_ids, w_slot):
    b1_3d = b1.reshape(E, 1, DFF)
    b2_3d = b2.reshape(E, 1, H)
    w_3d = w_slot.reshape(NUM_BLOCKS, 1, SEQ_PER_BLK)

    grid_spec = pltpu.PrefetchScalarGridSpec(
        num_scalar_prefetch=4,
        grid=(NT, NUM_BLOCKS),
        in_specs=[
            pl.BlockSpec((B, S, H), lambda t, b, nr, eb, si, pi: (0, 0, 0)),
            pl.BlockSpec((1, H, DFFT), lambda t, b, nr, eb, si, pi: (eb[b], 0, t)),
            pl.BlockSpec((1, 1, DFFT), lambda t, b, nr, eb, si, pi: (eb[b], 0, t)),
            pl.BlockSpec((1, DFFT, H), lambda t, b, nr, eb, si, pi: (eb[b], t, 0)),
            pl.BlockSpec((1, 1, H), lambda t, b, nr, eb, si, pi: (eb[b], 0, 0)),
            pl.BlockSpec((1, 1, SEQ_PER_BLK),
                         lambda t, b, nr, eb, si, pi: (b, 0, 0)),
        ],
        out_specs=pl.BlockSpec((NB * B + 1, S, H),
                               lambda t, b, nr, eb, si, pi: (0, 0, 0)),
    )
    out = pl.pallas_call(
        _ffn_body,
        grid_spec=grid_spec,
        out_shape=jax.ShapeDtypeStruct((NB * B + 1, S, H), jnp.float32),
    )(nreal, e_of_blk, seq_ids, pos_ids, x, W1, b1_3d, W2, b2_3d, w_3d)
    return out[:NB * B]


NK = NB * B


def _route_body(p_ref, pt_ref, beam_ref, gate_ref, seq_ref, pos_ref, w_ref,
                eob_ref, nreal_ref, loss_ref):
    i32 = jnp.int32
    f32 = jnp.float32
    p = p_ref[...]
    pt = pt_ref[...]
    cs = jnp.exp(jnp.log(p))
    cst = jnp.exp(jnp.log(pt))
    neg = jnp.float32(-jnp.inf)

    idc = jax.lax.broadcasted_iota(i32, (B, E), 1)
    m1c = jnp.max(cs, axis=1, keepdims=True)
    i1c = jnp.min(jnp.where(cs == m1c, idc, E), axis=1, keepdims=True)
    cs2 = jnp.where(idc == i1c, neg, cs)
    m2c = jnp.max(cs2, axis=1, keepdims=True)
    i2c = jnp.min(jnp.where(cs2 == m2c, idc, E), axis=1, keepdims=True)
    idr = jax.lax.broadcasted_iota(i32, (E, B), 0)
    m1r = jnp.max(cst, axis=0, keepdims=True)
    i1r = jnp.min(jnp.where(cst == m1r, idr, E), axis=0, keepdims=True)
    cst2 = jnp.where(idr == i1r, neg, cst)
    m2r = jnp.max(cst2, axis=0, keepdims=True)
    i2r = jnp.min(jnp.where(cst2 == m2r, idr, E), axis=0, keepdims=True)

    beam_ref[...] = jnp.concatenate([m1c, m2c], axis=1)
    gate_ref[...] = jnp.concatenate([i1c, i2c], axis=1)

    imp = jnp.sum(p, axis=0, keepdims=True)
    mean = jnp.sum(imp, axis=1, keepdims=True) / f32(E)
    var = jnp.sum((imp - mean) ** 2, axis=1, keepdims=True) / f32(E - 1)
    loss_ref[...] = var / (mean * mean)

    e_col = jnp.concatenate([i1c, i2c], axis=0)
    e_row = jnp.concatenate([i1r, i2r], axis=1)
    w_row = jnp.concatenate([m1r, m2r], axis=1)
    kr = jax.lax.broadcasted_iota(i32, (1, NK), 1)
    seq_row = kr % B
    pos_row = NB * (kr % B) + kr // B

    a_idx = jax.lax.broadcasted_iota(i32, (NK, NK), 0)
    b_idx = jax.lax.broadcasted_iota(i32, (NK, NK), 1)
    same_e = (e_row == e_col)
    rk_row = jnp.sum((same_e & (a_idx < b_idx)).astype(i32),
                     axis=0, keepdims=True)

    ec8 = jax.lax.broadcasted_iota(i32, (E, NK), 0)
    cnt8c = jnp.sum((e_row == ec8).astype(i32), axis=1, keepdims=True)
    er8 = jax.lax.broadcasted_iota(i32, (NK, E), 1)
    cnt8r = jnp.sum((e_col == er8).astype(i32), axis=0, keepdims=True)
    blk8c = (cnt8c + SEQ_PER_BLK - 1) // SEQ_PER_BLK
    blk8r = (cnt8r + SEQ_PER_BLK - 1) // SEQ_PER_BLK
    t8a = jax.lax.broadcasted_iota(i32, (E, E), 0)
    t8b = jax.lax.broadcasted_iota(i32, (E, E), 1)
    base8r = jnp.sum(jnp.where(t8a < t8b, blk8c, 0), axis=0,
                     keepdims=True)
    base8c = jnp.sum(jnp.where(t8b < t8a, blk8r, 0), axis=1,
                     keepdims=True)
    nreal = jnp.sum(blk8r, axis=1, keepdims=True)
    nreal_ref[...] = nreal

    base_of_k = jnp.sum(jnp.where(e_row == ec8, base8c, 0), axis=0,
                        keepdims=True)
    slot_row = SEQ_PER_BLK * base_of_k + rk_row

    nslot = (NUM_BLOCKS + 1) * SEQ_PER_BLK
    s_idx = jax.lax.broadcasted_iota(i32, (nslot, NK), 0)
    match = (slot_row == s_idx)
    found = jnp.sum(match.astype(i32), axis=1, keepdims=True)
    seq_ref[...] = jnp.sum(jnp.where(match, seq_row, 0), axis=1, keepdims=True)
    pos_ref[...] = jnp.where(
        found > 0,
        jnp.sum(jnp.where(match, pos_row, 0), axis=1, keepdims=True),
        TRASH_ROW)
    w_ref[...] = jnp.sum(jnp.where(match, w_row, jnp.float32(0.0)),
                         axis=1, keepdims=True)

    jb = jax.lax.broadcasted_iota(i32, (E, NUM_BLOCKS + 1), 1)
    eb8 = jax.lax.broadcasted_iota(i32, (E, NUM_BLOCKS + 1), 0)
    inb = (base8c <= jb) & (jb < base8c + blk8c)
    vale = jnp.sum(jnp.where(inb, eb8, 0), axis=0, keepdims=True)
    foundb = jnp.sum(inb.astype(i32), axis=0, keepdims=True)
    e_last = jnp.max(jnp.where(cnt8r > 0,
                               jax.lax.broadcasted_iota(i32, (1, E), 1), -1),
                     axis=1, keepdims=True)
    eob_ref[...] = jnp.where(foundb > 0, vale, e_last)


def _run_route(prob_gate):
    outs = pl.pallas_call(
        _route_body,
        out_shape=(
            jax.ShapeDtypeStruct((B, NB), jnp.float32),
            jax.ShapeDtypeStruct((B, NB), jnp.int32),
            jax.ShapeDtypeStruct(((NUM_BLOCKS + 1) * SEQ_PER_BLK, 1),
                                 jnp.int32),
            jax.ShapeDtypeStruct(((NUM_BLOCKS + 1) * SEQ_PER_BLK, 1),
                                 jnp.int32),
            jax.ShapeDtypeStruct(((NUM_BLOCKS + 1) * SEQ_PER_BLK, 1),
                                 jnp.float32),
            jax.ShapeDtypeStruct((1, NUM_BLOCKS + 1), jnp.int32),
            jax.ShapeDtypeStruct((1, 1), jnp.int32),
            jax.ShapeDtypeStruct((1, 1), jnp.float32),
        ),
    )(prob_gate, prob_gate.T)
    return outs


def kernel(x, W_gate, W1, b1, W2, b2):
    x_avg = jnp.sum(x, axis=1) / jnp.float32(x.shape[1])
    logits_gate = x_avg @ W_gate.T
    prob_gate = jax.nn.softmax(logits_gate, axis=-1)

    (beam2, gate2, seq16, pos16, w16, eob16, nreal2, loss2) = \
        _run_route(prob_gate)

    beam_scores = beam2.reshape(NB * B)
    expert_route = gate2.reshape(NB * B)[:, None]
    importance_loss = loss2.reshape(())

    seq_ids = seq16.reshape((NUM_BLOCKS + 1) * SEQ_PER_BLK)
    pos_ids = pos16.reshape((NUM_BLOCKS + 1) * SEQ_PER_BLK)
    w_slot = w16.reshape(NUM_BLOCKS + 1, SEQ_PER_BLK)[:NUM_BLOCKS].reshape(
        NUM_BLOCKS, 1, SEQ_PER_BLK)
    e_of_blk = eob16.reshape(NUM_BLOCKS + 1)[:NUM_BLOCKS]
    nreal = nreal2.reshape(1)

    output = _run_ffn(x, W1, b1, W2, b2, nreal, e_of_blk, seq_ids, pos_ids,
                      w_slot)
    return output, beam_scores, expert_route, importance_loss
